# baseline (speedup 1.0000x reference)
import jax
import jax.numpy as jnp
from jax.experimental import pallas as pl
from jax.experimental.pallas import tpu as pltpu

NUM_EMBED = 8192
N_HEADS = 4
D = 64
DH = D // N_HEADS
COMMIT = 0.25
BLK = 512


def _vq_block_kernel(x_ref, w_ref, waug_ref, q_ref, loss_ref,
                     c0_ref, c1_ref, c2_ref, c3_ref):
    code_refs = (c0_ref, c1_ref, c2_ref, c3_ref)
    x = x_ref[...]
    acc = jnp.zeros((BLK,), jnp.float32)
    for h in range(N_HEADS):
        xh = x[:, h * DH:(h + 1) * DH]
        W = w_ref[h]
        wsq = jnp.sum(W * W, axis=1)
        xw2 = jax.lax.dot_general(
            2.0 * xh, W, (((1,), (1,)), ((), ())),
            preferred_element_type=jnp.float32)
        logits = xw2 - wsq[None, :]
        e = jnp.exp(logits)
        qs = jax.lax.dot_general(
            e, waug_ref[h], (((1,), (0,)), ((), ())),
            preferred_element_type=jnp.float32)
        qh = qs[:, :DH] / qs[:, DH:DH + 1]
        q_ref[:, h * DH:(h + 1) * DH] = qh
        code = jnp.argmax(logits, axis=1).astype(jnp.int32)
        code_refs[h][...] = code.reshape(BLK, 1)
        diff = qh - xh
        acc = acc + jnp.sum(diff * diff, axis=1)
    loss_ref[...] = ((1.0 + COMMIT) / D * acc).reshape(BLK, 1)


def kernel(inputs, weights):
    b = inputs.shape[0]
    x = inputs.reshape(b, D)
    waug = jnp.concatenate(
        [weights, jnp.ones((N_HEADS, NUM_EMBED, 1), jnp.float32)], axis=2)
    grid = (b // BLK,)
    out_shapes = (
        jax.ShapeDtypeStruct((b, D), jnp.float32),
        jax.ShapeDtypeStruct((b, 1), jnp.float32),
    ) + tuple(jax.ShapeDtypeStruct((b, 1), jnp.int32) for _ in range(N_HEADS))
    out_specs = (
        pl.BlockSpec((BLK, D), lambda i: (i, 0)),
        pl.BlockSpec((BLK, 1), lambda i: (i, 0)),
    ) + tuple(pl.BlockSpec((BLK, 1), lambda i: (i, 0)) for _ in range(N_HEADS))
    outs = pl.pallas_call(
        _vq_block_kernel,
        grid=grid,
        in_specs=[
            pl.BlockSpec((BLK, D), lambda i: (i, 0)),
            pl.BlockSpec((N_HEADS, NUM_EMBED, DH), lambda i: (0, 0, 0)),
            pl.BlockSpec((N_HEADS, NUM_EMBED, DH + 1), lambda i: (0, 0, 0)),
        ],
        out_specs=out_specs,
        out_shape=out_shapes,
        compiler_params=pltpu.CompilerParams(
            dimension_semantics=("arbitrary",),
        ),
    )(x, weights, waug)
    quantized = outs[0].reshape(inputs.shape)
    loss = outs[1].reshape(b)
    codes = jnp.concatenate(outs[2:], axis=1)
    return (loss, quantized, codes)


# speedup vs baseline: 1.0175x; 1.0175x over previous
import jax
import jax.numpy as jnp
from jax.experimental import pallas as pl
from jax.experimental.pallas import tpu as pltpu

NUM_EMBED = 8192
N_HEADS = 4
D = 64
DH = D // N_HEADS
COMMIT = 0.25
BLK = 512


def _vq_block_kernel(x_ref, w_ref, waug_ref, q_ref, loss_ref,
                     c0_ref, c1_ref, c2_ref, c3_ref, wsq_ref):
    code_refs = (c0_ref, c1_ref, c2_ref, c3_ref)
    x = x_ref[...]
    acc = jnp.zeros((BLK,), jnp.float32)

    @pl.when(pl.program_id(0) == 0)
    def _():
        w_all = w_ref[...]
        wsq_ref[...] = jnp.sum(w_all * w_all, axis=2)

    for h in range(N_HEADS):
        xh = x[:, h * DH:(h + 1) * DH]
        W = w_ref[h]
        wsq = wsq_ref[h]
        xw2 = jax.lax.dot_general(
            2.0 * xh, W, (((1,), (1,)), ((), ())),
            preferred_element_type=jnp.float32)
        logits = xw2 - wsq[None, :]
        e = jnp.exp(logits)
        qs = jax.lax.dot_general(
            e, waug_ref[h], (((1,), (0,)), ((), ())),
            preferred_element_type=jnp.float32)
        qh = qs[:, :DH] / qs[:, DH:DH + 1]
        q_ref[:, h * DH:(h + 1) * DH] = qh
        code = jnp.argmax(logits, axis=1).astype(jnp.int32)
        code_refs[h][...] = code.reshape(BLK, 1)
        diff = qh - xh
        acc = acc + jnp.sum(diff * diff, axis=1)
    loss_ref[...] = ((1.0 + COMMIT) / D * acc).reshape(BLK, 1)


def kernel(inputs, weights):
    b = inputs.shape[0]
    x = inputs.reshape(b, D)
    waug = jnp.concatenate(
        [weights, jnp.ones((N_HEADS, NUM_EMBED, 1), jnp.float32)], axis=2)
    grid = (b // BLK,)
    out_shapes = (
        jax.ShapeDtypeStruct((b, D), jnp.float32),
        jax.ShapeDtypeStruct((b, 1), jnp.float32),
    ) + tuple(jax.ShapeDtypeStruct((b, 1), jnp.int32) for _ in range(N_HEADS))
    out_specs = (
        pl.BlockSpec((BLK, D), lambda i: (i, 0)),
        pl.BlockSpec((BLK, 1), lambda i: (i, 0)),
    ) + tuple(pl.BlockSpec((BLK, 1), lambda i: (i, 0)) for _ in range(N_HEADS))
    outs = pl.pallas_call(
        _vq_block_kernel,
        grid=grid,
        in_specs=[
            pl.BlockSpec((BLK, D), lambda i: (i, 0)),
            pl.BlockSpec((N_HEADS, NUM_EMBED, DH), lambda i: (0, 0, 0)),
            pl.BlockSpec((N_HEADS, NUM_EMBED, DH + 1), lambda i: (0, 0, 0)),
        ],
        out_specs=out_specs,
        out_shape=out_shapes,
        scratch_shapes=[pltpu.VMEM((N_HEADS, NUM_EMBED), jnp.float32)],
        compiler_params=pltpu.CompilerParams(
            dimension_semantics=("arbitrary",),
        ),
    )(x, weights, waug)
    quantized = outs[0].reshape(inputs.shape)
    loss = outs[1].reshape(b)
    codes = jnp.concatenate(outs[2:], axis=1)
    return (loss, quantized, codes)
